# baseline (speedup 1.0000x reference)
import jax
import jax.numpy as jnp
from jax import lax
from jax.experimental import pallas as pl
from jax.experimental.pallas import tpu as pltpu
from jax.experimental.pallas import tpu_sc as plsc

_INFO = plsc.get_sparse_core_info()
_NC = _INFO.num_cores
_NS = _INFO.num_subcores
_NW = _NC * _NS

_BATCH = 16384
_DIM = 64
_LANES = 128
_BLK = 128
_BLK_PER_W = _BATCH // (_NW * _BLK)
_PER_W = _BLK_PER_W * _BLK
_NV = _BLK // 16


def _gather_body(idx_hbm, pairs_hbm, out_hbm, idx_v, pidx_v, par_v, rows_v,
                 blk_v, sems):
    wid = lax.axis_index("s") * _NC + lax.axis_index("c")
    base = wid * _PER_W

    pltpu.sync_copy(idx_hbm.at[pl.ds(base, _PER_W)], idx_v)
    for k in range(_BLK_PER_W):
        for g in range(_NV):
            t = k * _NV + g
            v = idx_v[pl.ds(t * 16, 16)]
            pidx_v[k, pl.ds(g * 16, 16)] = lax.shift_right_logical(v, 1)
            par_v[k, pl.ds(g * 16, 16)] = lax.shift_left(
                lax.bitwise_and(v, 1), 6)

    copies = [
        pltpu.async_copy(pairs_hbm.at[pidx_v.at[k]], rows_v.at[k], sems.at[k])
        for k in range(_BLK_PER_W)
    ]

    for k in range(_BLK_PER_W):
        copies[k].wait()
        rows_k = rows_v.at[k]
        for g in range(_NV):
            rowv = lax.iota(jnp.int32, 16) + (g * 16)
            p64 = par_v[k, pl.ds(g * 16, 16)]
            for d in range(_DIM):
                colv = p64 + d
                blk_v[d, pl.ds(g * 16, 16)] = plsc.load_gather(
                    rows_k, [rowv, colv])
        pltpu.sync_copy(
            blk_v, out_hbm.at[:, pl.ds((base + k * _BLK), _BLK)])


@jax.jit
def _gather(idx, pairs):
    mesh = plsc.VectorSubcoreMesh(core_axis_name="c", subcore_axis_name="s")
    run = pl.kernel(
        _gather_body,
        out_type=jax.ShapeDtypeStruct((_DIM, _BATCH), jnp.float32),
        mesh=mesh,
        scratch_types=[
            pltpu.VMEM((_PER_W,), jnp.int32),
            pltpu.VMEM((_BLK_PER_W, _BLK), jnp.int32),
            pltpu.VMEM((_BLK_PER_W, _BLK), jnp.int32),
            pltpu.VMEM((_BLK_PER_W, _BLK, _LANES), jnp.float32),
            pltpu.VMEM((_DIM, _BLK), jnp.float32),
            pltpu.SemaphoreType.DMA((_BLK_PER_W,)),
        ],
        compiler_params=pltpu.CompilerParams(
            use_tc_tiling_on_sc=True,
            disable_bounds_checks=True,
            needs_layout_passes=False,
        ),
    )
    return run(idx, pairs)


def kernel(indices, latents):
    idx = indices.astype(jnp.int32)
    pairs = latents.reshape(latents.shape[0] // 2, 2 * _DIM)
    out_t = _gather(idx, pairs)
    return out_t.T.reshape(_BATCH, 1, _DIM)


# speedup vs baseline: 1.3148x; 1.3148x over previous
import jax
import jax.numpy as jnp
from jax import lax
from jax.experimental import pallas as pl
from jax.experimental.pallas import tpu as pltpu
from jax.experimental.pallas import tpu_sc as plsc

_INFO = plsc.get_sparse_core_info()
_NC = _INFO.num_cores
_NS = _INFO.num_subcores
_NW = _NC * _NS

_BATCH = 16384
_DIM = 64
_WIDE = 128
_BLK = 128
_BLK_PER_W = _BATCH // (_NW * _BLK)
_PER_W = _BLK_PER_W * _BLK


def _gather_body(idx_hbm, wide_hbm, out_hbm, idx_v, rows_v, sems):
    wid = lax.axis_index("s") * _NC + lax.axis_index("c")
    base = wid * _PER_W

    for k in range(_BLK_PER_W):
        pltpu.sync_copy(idx_hbm.at[pl.ds(base + k * _BLK, _BLK)], idx_v.at[k])
    copies = [
        pltpu.async_copy(wide_hbm.at[idx_v.at[k]], rows_v.at[k], sems.at[k])
        for k in range(_BLK_PER_W)
    ]
    for k in range(_BLK_PER_W):
        copies[k].wait()
        pltpu.sync_copy(
            rows_v.at[k], out_hbm.at[pl.ds(base + k * _BLK, _BLK), :])


@jax.jit
def _gather(idx, wide):
    mesh = plsc.VectorSubcoreMesh(core_axis_name="c", subcore_axis_name="s")
    run = pl.kernel(
        _gather_body,
        out_type=jax.ShapeDtypeStruct((_BATCH, _WIDE), jnp.float32),
        mesh=mesh,
        scratch_types=[
            pltpu.VMEM((_BLK_PER_W, _BLK), jnp.int32),
            pltpu.VMEM((_BLK_PER_W, _BLK, _WIDE), jnp.float32),
            pltpu.SemaphoreType.DMA((_BLK_PER_W,)),
        ],
        compiler_params=pltpu.CompilerParams(
            use_tc_tiling_on_sc=True,
            disable_bounds_checks=True,
        ),
    )
    return run(idx, wide)


def kernel(indices, latents):
    idx = indices.astype(jnp.int32)
    table = latents.reshape(latents.shape[0], _DIM)
    wide = jnp.concatenate(
        [table, jnp.zeros_like(table)], axis=1)
    out128 = _gather(idx, wide)
    return out128[:, :_DIM].reshape(_BATCH, 1, _DIM)


# speedup vs baseline: 1.3148x; 1.0000x over previous
import jax
import jax.numpy as jnp
from jax import lax
from jax.experimental import pallas as pl
from jax.experimental.pallas import tpu as pltpu
from jax.experimental.pallas import tpu_sc as plsc

_INFO = plsc.get_sparse_core_info()
_NC = _INFO.num_cores
_NS = _INFO.num_subcores
_NW = _NC * _NS

_BATCH = 16384
_DIM = 64
_WIDE = 128
_BLK = 128
_BLK_PER_W = _BATCH // (_NW * _BLK)
_PER_W = _BLK_PER_W * _BLK


def _gather_body(idx_hbm, wide_hbm, out_hbm, idx_v, rows_v, sems):
    wid = lax.axis_index("s") * _NC + lax.axis_index("c")
    base = wid * _PER_W

    for k in range(_BLK_PER_W):
        pltpu.sync_copy(idx_hbm.at[pl.ds(base + k * _BLK, _BLK)], idx_v.at[k])
    copies = [
        pltpu.async_copy(wide_hbm.at[idx_v.at[k]], rows_v.at[k], sems.at[k])
        for k in range(_BLK_PER_W)
    ]
    for k in range(_BLK_PER_W):
        copies[k].wait()
        pltpu.sync_copy(
            rows_v.at[k], out_hbm.at[pl.ds(base + k * _BLK, _BLK), :])


@jax.jit
def _gather(idx, wide):
    mesh = plsc.VectorSubcoreMesh(core_axis_name="c", subcore_axis_name="s")
    run = pl.kernel(
        _gather_body,
        out_type=jax.ShapeDtypeStruct((_BATCH, _WIDE), jnp.float32),
        mesh=mesh,
        scratch_types=[
            pltpu.VMEM((_BLK_PER_W, _BLK), jnp.int32),
            pltpu.VMEM((_BLK_PER_W, _BLK, _WIDE), jnp.float32),
            pltpu.SemaphoreType.DMA((_BLK_PER_W,)),
        ],
        compiler_params=pltpu.CompilerParams(
            use_tc_tiling_on_sc=True,
            disable_bounds_checks=True,
        ),
    )
    return run(idx, wide)


def kernel(indices, latents):
    idx = indices.astype(jnp.int32)
    table = latents.reshape(latents.shape[0], _DIM)
    wide = jnp.pad(
        latents, ((0, 0), (0, 0), (0, _WIDE - _DIM))
    ).reshape(latents.shape[0], _WIDE)
    out128 = _gather(idx, wide)
    return out128[:, :_DIM].reshape(_BATCH, 1, _DIM)


# speedup vs baseline: 1.3326x; 1.0135x over previous
import jax
import jax.numpy as jnp
from jax import lax
from jax.experimental import pallas as pl
from jax.experimental.pallas import tpu as pltpu
from jax.experimental.pallas import tpu_sc as plsc

_INFO = plsc.get_sparse_core_info()
_NC = _INFO.num_cores
_NS = _INFO.num_subcores
_NW = _NC * _NS

_BATCH = 16384
_DIM = 64
_WIDE = 128
_BLK = 128
_BLK_PER_W = _BATCH // (_NW * _BLK)
_PER_W = _BLK_PER_W * _BLK


def _gather_body(idx_hbm, wide_hbm, out_hbm, idx_v, rows_v, sems):
    wid = lax.axis_index("s") * _NC + lax.axis_index("c")
    base = wid * _PER_W

    pltpu.sync_copy(idx_hbm.at[pl.ds(base, _PER_W)], idx_v)
    copies = [
        pltpu.async_copy(
            wide_hbm.at[idx_v.at[pl.ds(k * _BLK, _BLK)]],
            rows_v.at[k],
            sems.at[k],
        )
        for k in range(_BLK_PER_W)
    ]
    for k in range(_BLK_PER_W):
        copies[k].wait()
        pltpu.sync_copy(
            rows_v.at[k], out_hbm.at[pl.ds(base + k * _BLK, _BLK), :])


@jax.jit
def _gather(idx, wide):
    mesh = plsc.VectorSubcoreMesh(core_axis_name="c", subcore_axis_name="s")
    run = pl.kernel(
        _gather_body,
        out_type=jax.ShapeDtypeStruct((_BATCH, _WIDE), jnp.float32),
        mesh=mesh,
        scratch_types=[
            pltpu.VMEM((_PER_W,), jnp.int32),
            pltpu.VMEM((_BLK_PER_W, _BLK, _WIDE), jnp.float32),
            pltpu.SemaphoreType.DMA((_BLK_PER_W,)),
        ],
        compiler_params=pltpu.CompilerParams(
            use_tc_tiling_on_sc=True,
            disable_bounds_checks=True,
        ),
    )
    return run(idx, wide)


def kernel(indices, latents):
    idx = indices.astype(jnp.int32)
    table = latents.reshape(latents.shape[0], _DIM)
    wide = jnp.pad(
        latents, ((0, 0), (0, 0), (0, _WIDE - _DIM))
    ).reshape(latents.shape[0], _WIDE)
    out128 = _gather(idx, wide)
    return out128[:, :_DIM].reshape(_BATCH, 1, _DIM)
